# baseline (speedup 1.0000x reference)
import jax
import jax.numpy as jnp
from jax.experimental import pallas as pl
from jax.experimental.pallas import tpu as pltpu

N = 5000
D = 12544
H = 1024
NC = 4
NB = 12
OW = 128

BN = 200
NN = N // BN
NCH = 14
CH = D // NCH
NSTEPS = NCH + NN - 1


def _stage2(h1, b1_ref, w2_ref, b2_ref, w34_ref, b34_ref):
    h = jnp.maximum(h1 + b1_ref[...], 0.0)
    h2 = jax.lax.dot_general(
        h, w2_ref[...], (((1,), (0,)), ((), ())),
        preferred_element_type=jnp.float32) + b2_ref[...]
    h2 = jnp.maximum(h2, 0.0)
    o = jax.lax.dot_general(
        h2, w34_ref[...], (((1,), (0,)), ((), ())),
        preferred_element_type=jnp.float32) + b34_ref[...]
    col = jax.lax.broadcasted_iota(jnp.int32, o.shape, 1)
    is_cls = col < NC
    neg = jnp.where(is_cls, o, -1e30)
    m = jnp.max(neg, axis=1, keepdims=True)
    e = jnp.where(is_cls, jnp.exp(o - m), 0.0)
    sm = jnp.sum(e, axis=1, keepdims=True)
    return jnp.where(is_cls, e / sm, o)


def _body(x_ref, w1_ref, w2_ref, b1_ref, b2_ref, w34_ref, b34_ref,
          out_ref, w1b_ref, acc_ref):
    s = pl.program_id(0)

    @pl.when(s < NCH)
    def _cast():
        j = jnp.minimum(s, NCH - 1)
        cols = pl.ds(j * CH, CH)
        w1b = w1_ref[...].astype(jnp.bfloat16)
        w1b_ref[cols, :] = w1b
        part = jax.lax.dot_general(
            x_ref[:, cols], w1b, (((1,), (0,)), ((), ())),
            preferred_element_type=jnp.float32)

        @pl.when(s == 0)
        def _():
            acc_ref[...] = part

        @pl.when(s > 0)
        def _():
            acc_ref[...] += part

    @pl.when(s >= NCH)
    def _compute():
        tile = s - NCH + 1

        @pl.when(s == NCH)
        def _tile0():
            out_ref[pl.ds(0, BN), :] = _stage2(
                acc_ref[...], b1_ref, w2_ref, b2_ref, w34_ref, b34_ref)

        h1 = jax.lax.dot_general(
            x_ref[...], w1b_ref[...], (((1,), (0,)), ((), ())),
            preferred_element_type=jnp.float32)
        out_ref[pl.ds(tile * BN, BN), :] = _stage2(
            h1, b1_ref, w2_ref, b2_ref, w34_ref, b34_ref)


def kernel(feature_vectors, W1, b1, W2, b2, W3, b3, W4, b4):
    f32, bf16 = jnp.float32, jnp.bfloat16
    W34 = jnp.zeros((H, OW), f32).at[:, :NC].set(W3).at[:, NC:NC + NB].set(W4)
    b34 = jnp.zeros((1, OW), f32).at[0, :NC].set(b3).at[0, NC:NC + NB].set(b4)

    def _xmap(s):
        return (jnp.where(s < NCH, 0, jnp.clip(s - NCH + 1, 0, NN - 1)), 0)

    out = pl.pallas_call(
        _body,
        grid=(NSTEPS,),
        in_specs=[
            pl.BlockSpec((BN, D), _xmap),
            pl.BlockSpec((CH, H), lambda s: (jnp.minimum(s, NCH - 1), 0)),
            pl.BlockSpec((H, H), lambda s: (0, 0)),
            pl.BlockSpec((1, H), lambda s: (0, 0)),
            pl.BlockSpec((1, H), lambda s: (0, 0)),
            pl.BlockSpec((H, OW), lambda s: (0, 0)),
            pl.BlockSpec((1, OW), lambda s: (0, 0)),
        ],
        out_specs=pl.BlockSpec((N, OW), lambda s: (0, 0)),
        out_shape=jax.ShapeDtypeStruct((N, OW), f32),
        scratch_shapes=[pltpu.VMEM((D, H), bf16),
                        pltpu.VMEM((BN, H), f32)],
        compiler_params=pltpu.CompilerParams(
            dimension_semantics=("arbitrary",),
            vmem_limit_bytes=62 * 1024 * 1024,
        ),
    )(feature_vectors, W1, W2.astype(bf16),
      b1.reshape(1, H), b2.reshape(1, H), W34.astype(bf16), b34)

    return out[:, :NC], out[:, NC:NC + NB]
